# baseline (speedup 1.0000x reference)
import functools
import math

import jax
import jax.numpy as jnp
from jax import lax
from jax.experimental import pallas as pl
from jax.experimental.pallas import tpu as pltpu
from jax.experimental.pallas import tpu_sc as plsc

D_MODEL = 64
BATCH = 16384
DENSE_DIM = 16
SCALE = 1.0 / math.sqrt(4.0)
PITCH = 128

NUM_CORES = 2
NUM_SUBCORES = 16
NW = NUM_CORES * NUM_SUBCORES
B_W = BATCH // NW
L = 16
BN = 4096


def _tc_linearize(tblT, eye):
    v = tblT.shape[1]
    nb = (v + BN - 1) // BN

    def body(t_ref, eye_ref, out_ref):
        rows = lax.dot_general(
            t_ref[...], eye_ref[...], (((0,), (0,)), ((), ())),
            preferred_element_type=jnp.float32)
        padded = jnp.concatenate(
            [rows, jnp.zeros((BN, PITCH - D_MODEL), jnp.float32)], axis=1)
        out_ref[...] = padded.reshape(BN * PITCH)

    return pl.pallas_call(
        body,
        grid=(nb,),
        in_specs=[
            pl.BlockSpec((D_MODEL, BN), lambda i: (0, i)),
            pl.BlockSpec((D_MODEL, D_MODEL), lambda i: (0, 0)),
        ],
        out_specs=pl.BlockSpec((BN * PITCH,), lambda i: (i,)),
        out_shape=jax.ShapeDtypeStruct((nb * BN * PITCH,), jnp.float32),
    )(tblT, eye)


def _sc_gather_sum(user_id, item_id, category, eu_flat, ei_flat, ec_flat):
    mesh = plsc.VectorSubcoreMesh(core_axis_name="c", subcore_axis_name="s")

    @functools.partial(
        pl.kernel,
        mesh=mesh,
        compiler_params=pltpu.CompilerParams(use_tc_tiling_on_sc=True),
        out_type=jax.ShapeDtypeStruct((BATCH * D_MODEL,), jnp.float32),
        scratch_types=[
            pltpu.VMEM((B_W,), jnp.int32),
            pltpu.VMEM((B_W,), jnp.int32),
            pltpu.VMEM((B_W,), jnp.int32),
            pltpu.VMEM((B_W * D_MODEL,), jnp.float32),
            pltpu.VMEM((B_W * D_MODEL,), jnp.float32),
            pltpu.SemaphoreType.DMA,
        ],
    )
    def k(eu_hbm, ei_hbm, ec_hbm, uid_hbm, iid_hbm, cat_hbm, out_hbm,
          idx_u, idx_i, idx_c, acc, buf, sem):
        wid = lax.axis_index("s") * NUM_CORES + lax.axis_index("c")
        base = wid * B_W
        pltpu.sync_copy(uid_hbm.at[pl.ds(base, B_W)], idx_u)
        pltpu.sync_copy(iid_hbm.at[pl.ds(base, B_W)], idx_i)
        pltpu.sync_copy(cat_hbm.at[pl.ds(base, B_W)], idx_c)

        def fire(tbl, idx, rows):
            def fire_group(g, _):
                iv = idx[pl.ds(g * L, L)]
                for l in range(L):
                    pltpu.make_async_copy(
                        tbl.at[pl.ds(iv[l] * PITCH, D_MODEL)],
                        rows.at[pl.ds((g * L + l) * D_MODEL, D_MODEL)],
                        sem,
                    ).start()
                return 0

            lax.fori_loop(0, B_W // L, fire_group, 0)

        def drain(rows):
            pltpu.make_async_copy(
                eu_hbm.at[pl.ds(0, B_W * D_MODEL)], rows, sem).wait()

        def accumulate(v, _):
            sl = pl.ds(v * L, L)
            acc[sl] = acc[sl] + buf[sl]
            return 0

        fire(eu_hbm, idx_u, acc)
        fire(ei_hbm, idx_i, buf)
        drain(acc)
        drain(buf)
        lax.fori_loop(0, B_W * D_MODEL // L, accumulate, 0)
        fire(ec_hbm, idx_c, buf)
        drain(buf)
        lax.fori_loop(0, B_W * D_MODEL // L, accumulate, 0)
        pltpu.sync_copy(acc, out_hbm.at[pl.ds(base * D_MODEL, B_W * D_MODEL)])

    return k(eu_flat, ei_flat, ec_flat, user_id, item_id, category)


def _tc_combine(gsum, dense_0, W_dense, uid2, iid2, cat2, eu0, ei0, ec0):
    BM = 1024

    def body(gsum_ref, dense_ref, w_ref, uid_ref, iid_ref, cat_ref,
             eu0_ref, ei0_ref, ec0_ref, out_ref):
        proj = jnp.dot(dense_ref[...], w_ref[...],
                       preferred_element_type=jnp.float32)
        mu = (uid_ref[...] == 0).astype(jnp.float32)
        mi = (iid_ref[...] == 0).astype(jnp.float32)
        mc = (cat_ref[...] == 0).astype(jnp.float32)
        corr = (mu * eu0_ref[...] + mi * ei0_ref[...] + mc * ec0_ref[...])
        out_ref[...] = (gsum_ref[...] + proj - corr) * SCALE

    row_spec = pl.BlockSpec((1, D_MODEL), lambda i: (0, 0))
    return pl.pallas_call(
        body,
        grid=(BATCH // BM,),
        in_specs=[
            pl.BlockSpec((BM, D_MODEL), lambda i: (i, 0)),
            pl.BlockSpec((BM, DENSE_DIM), lambda i: (i, 0)),
            pl.BlockSpec((DENSE_DIM, D_MODEL), lambda i: (0, 0)),
            pl.BlockSpec((BM, 1), lambda i: (i, 0)),
            pl.BlockSpec((BM, 1), lambda i: (i, 0)),
            pl.BlockSpec((BM, 1), lambda i: (i, 0)),
            row_spec,
            row_spec,
            row_spec,
        ],
        out_specs=pl.BlockSpec((BM, D_MODEL), lambda i: (i, 0)),
        out_shape=jax.ShapeDtypeStruct((BATCH, D_MODEL), jnp.float32),
    )(gsum, dense_0, W_dense, uid2, iid2, cat2, eu0, ei0, ec0)


def kernel(user_id, item_id, category, dense_0, E_user, E_item, E_cat,
           W_dense):
    eye = jnp.eye(D_MODEL, dtype=jnp.float32)
    eu_flat = _tc_linearize(E_user.T, eye)
    ei_flat = _tc_linearize(E_item.T, eye)
    ec_flat = _tc_linearize(E_cat.T, eye)
    gflat = _sc_gather_sum(user_id, item_id, category,
                           eu_flat, ei_flat, ec_flat)
    gsum = gflat.reshape(BATCH, D_MODEL)
    return _tc_combine(
        gsum, dense_0, W_dense,
        user_id.reshape(BATCH, 1), item_id.reshape(BATCH, 1),
        category.reshape(BATCH, 1),
        lax.slice(E_user, (0, 0), (1, D_MODEL)),
        lax.slice(E_item, (0, 0), (1, D_MODEL)),
        lax.slice(E_cat, (0, 0), (1, D_MODEL)),
    )


# speedup vs baseline: 1.0512x; 1.0512x over previous
import functools
import math

import jax
import jax.numpy as jnp
from jax import lax
from jax.experimental import pallas as pl
from jax.experimental.pallas import tpu as pltpu
from jax.experimental.pallas import tpu_sc as plsc

D_MODEL = 64
BATCH = 16384
DENSE_DIM = 16
SCALE = 1.0 / math.sqrt(4.0)

NUM_CORES = 2
NUM_SUBCORES = 16
NW = NUM_CORES * NUM_SUBCORES
B_W = BATCH // NW
B_H = B_W // 2
L = 16


def _sc_gather_sum(user_id, item_id, category, E_user, E_item, E_cat):
    mesh = plsc.VectorSubcoreMesh(core_axis_name="c", subcore_axis_name="s")

    @functools.partial(
        pl.kernel,
        mesh=mesh,
        compiler_params=pltpu.CompilerParams(use_tc_tiling_on_sc=True),
        out_type=jax.ShapeDtypeStruct((BATCH, D_MODEL), jnp.float32),
        scratch_types=[
            pltpu.VMEM((B_W,), jnp.int32),
            pltpu.VMEM((B_W,), jnp.int32),
            pltpu.VMEM((B_W,), jnp.int32),
            pltpu.VMEM((B_H, D_MODEL), jnp.float32),
            pltpu.VMEM((B_H, D_MODEL), jnp.float32),
            pltpu.SemaphoreType.DMA,
        ],
    )
    def k(eu_hbm, ei_hbm, ec_hbm, uid_hbm, iid_hbm, cat_hbm, out_hbm,
          idx_u, idx_i, idx_c, acc, buf, sem):
        wid = lax.axis_index("s") * NUM_CORES + lax.axis_index("c")
        base = wid * B_W
        pltpu.sync_copy(uid_hbm.at[pl.ds(base, B_W)], idx_u)
        pltpu.sync_copy(iid_hbm.at[pl.ds(base, B_W)], idx_i)
        pltpu.sync_copy(cat_hbm.at[pl.ds(base, B_W)], idx_c)

        def fire(tbl, idx, rows, h):
            def fire_group(g, _):
                iv = idx[pl.ds(h * B_H + g * L, L)]
                for l in range(L):
                    pltpu.make_async_copy(
                        tbl.at[iv[l]], rows.at[g * L + l], sem).start()
                return 0

            lax.fori_loop(0, B_H // L, fire_group, 0)

        def drain(rows):
            pltpu.make_async_copy(
                out_hbm.at[pl.ds(0, B_H)], rows, sem).wait()

        def accumulate(r, _):
            for dsub in range(D_MODEL // L):
                sl = pl.ds(dsub * L, L)
                acc[r, sl] = acc[r, sl] + buf[r, sl]
            return 0

        for h in range(B_W // B_H):
            fire(eu_hbm, idx_u, acc, h)
            fire(ei_hbm, idx_i, buf, h)
            drain(acc)
            drain(buf)
            lax.fori_loop(0, B_H, accumulate, 0)
            fire(ec_hbm, idx_c, buf, h)
            drain(buf)
            lax.fori_loop(0, B_H, accumulate, 0)
            pltpu.sync_copy(acc, out_hbm.at[pl.ds(base + h * B_H, B_H)])

    return k(E_user, E_item, E_cat, user_id, item_id, category)


def _tc_combine(gsum, dense_0, W_dense, uid2, iid2, cat2, eu0, ei0, ec0):
    BM = 1024

    def body(gsum_ref, dense_ref, w_ref, uid_ref, iid_ref, cat_ref,
             eu0_ref, ei0_ref, ec0_ref, out_ref):
        proj = jnp.dot(dense_ref[...], w_ref[...],
                       preferred_element_type=jnp.float32)
        mu = (uid_ref[...] == 0).astype(jnp.float32)
        mi = (iid_ref[...] == 0).astype(jnp.float32)
        mc = (cat_ref[...] == 0).astype(jnp.float32)
        corr = (mu * eu0_ref[...] + mi * ei0_ref[...] + mc * ec0_ref[...])
        out_ref[...] = (gsum_ref[...] + proj - corr) * SCALE

    row_spec = pl.BlockSpec((1, D_MODEL), lambda i: (0, 0))
    return pl.pallas_call(
        body,
        grid=(BATCH // BM,),
        in_specs=[
            pl.BlockSpec((BM, D_MODEL), lambda i: (i, 0)),
            pl.BlockSpec((BM, DENSE_DIM), lambda i: (i, 0)),
            pl.BlockSpec((DENSE_DIM, D_MODEL), lambda i: (0, 0)),
            pl.BlockSpec((BM, 1), lambda i: (i, 0)),
            pl.BlockSpec((BM, 1), lambda i: (i, 0)),
            pl.BlockSpec((BM, 1), lambda i: (i, 0)),
            row_spec,
            row_spec,
            row_spec,
        ],
        out_specs=pl.BlockSpec((BM, D_MODEL), lambda i: (i, 0)),
        out_shape=jax.ShapeDtypeStruct((BATCH, D_MODEL), jnp.float32),
    )(gsum, dense_0, W_dense, uid2, iid2, cat2, eu0, ei0, ec0)


def kernel(user_id, item_id, category, dense_0, E_user, E_item, E_cat,
           W_dense):
    gsum = _sc_gather_sum(user_id, item_id, category, E_user, E_item, E_cat)
    return _tc_combine(
        gsum, dense_0, W_dense,
        user_id.reshape(BATCH, 1), item_id.reshape(BATCH, 1),
        category.reshape(BATCH, 1),
        lax.slice(E_user, (0, 0), (1, D_MODEL)),
        lax.slice(E_item, (0, 0), (1, D_MODEL)),
        lax.slice(E_cat, (0, 0), (1, D_MODEL)),
    )
